# baseline (speedup 1.0000x reference)
import jax
import jax.numpy as jnp
from jax.experimental import pallas as pl
from jax.experimental.pallas import tpu as pltpu

_N = 1024
_T = 4
_IN_DIM = 128
_C = 128
_H = 4
_LN2 = 0.6931471805599453


def _gat_kernel(x_ref, adj_ref, w_ref, asrc_ref, adst_ref, bias_ref,
                out_ref, logw_ref):
    h = pl.program_id(1)

    @pl.when(h == 0)
    def _build_logw():
        adjf = adj_ref[...].astype(jnp.float32)
        row_i = jax.lax.broadcasted_iota(jnp.int32, (_N, _N), 0)
        col_j = jax.lax.broadcasted_iota(jnp.int32, (_N, _N), 1)
        count = adjf + (row_i == col_j).astype(jnp.float32)
        logw_ref[...] = jnp.where(count == 0.0, -1e30, count - 1.0)

    x_t = x_ref[0]
    w_h = w_ref[...]
    hfeat = jax.lax.dot_general(
        x_t, w_h, (((1,), (1,)), ((), ())),
        preferred_element_type=jnp.float32)
    asrc_col = jax.lax.dot_general(
        hfeat, asrc_ref[0], (((1,), (1,)), ((), ())),
        preferred_element_type=jnp.float32)
    adst_row = jax.lax.dot_general(
        adst_ref[0], hfeat, (((1,), (1,)), ((), ())),
        preferred_element_type=jnp.float32)

    s = asrc_col + adst_row
    s = jnp.maximum(s, 0.2 * s)
    e = s + logw_ref[...]
    amax = jnp.max(e, axis=0, keepdims=True)
    p = jnp.exp2(e - amax)

    contrib = jax.lax.dot_general(
        p, hfeat, (((0,), (0,)), ((), ())),
        preferred_element_type=jnp.float32)
    ones = jnp.ones((_N, 1), dtype=jnp.float32)
    denom = jax.lax.dot_general(
        p, ones, (((0,), (0,)), ((), ())),
        preferred_element_type=jnp.float32)
    contrib = contrib * ((1.0 / _H) / (denom + 1e-16))

    @pl.when(h == 0)
    def _init():
        out_ref[0] = bias_ref[...] + contrib

    @pl.when(h != 0)
    def _acc():
        out_ref[0] = out_ref[0] + contrib


def kernel(x, node_matrix, W, att_src, att_dst, bias):
    x_t_major = jnp.transpose(x, (1, 0, 2))
    bias2d = bias.reshape(1, _C)
    log2e = jnp.float32(1.4426950408889634)
    out = pl.pallas_call(
        _gat_kernel,
        grid=(_T, _H),
        in_specs=[
            pl.BlockSpec((1, _N, _IN_DIM), lambda t, h: (t, 0, 0)),
            pl.BlockSpec((_N, _N), lambda t, h: (0, 0)),
            pl.BlockSpec((_C, _IN_DIM), lambda t, h: (h, 0)),
            pl.BlockSpec((1, 1, _C), lambda t, h: (h, 0, 0)),
            pl.BlockSpec((1, 1, _C), lambda t, h: (h, 0, 0)),
            pl.BlockSpec((1, _C), lambda t, h: (0, 0)),
        ],
        out_specs=pl.BlockSpec((1, _N, _C), lambda t, h: (t, 0, 0)),
        out_shape=jax.ShapeDtypeStruct((_T, _N, _C), jnp.float32),
        scratch_shapes=[pltpu.VMEM((_N, _N), jnp.float32)],
        compiler_params=pltpu.CompilerParams(
            dimension_semantics=("parallel", "arbitrary")),
    )(x_t_major, node_matrix, W,
      (att_src * log2e).reshape(_H, 1, _C),
      (att_dst * log2e).reshape(_H, 1, _C), bias2d)
    return jnp.transpose(out, (1, 0, 2))


# speedup vs baseline: 1.0284x; 1.0284x over previous
import jax
import jax.numpy as jnp
from jax.experimental import pallas as pl
from jax.experimental.pallas import tpu as pltpu

_N = 1024
_T = 4
_IN_DIM = 128
_C = 128
_H = 4
_LN2 = 0.6931471805599453


def _gat_kernel(x_ref, adj_ref, w_ref, asrc_ref, adst_ref, bias_ref,
                out_ref, logw_ref):
    t = pl.program_id(0)
    h = pl.program_id(1)

    @pl.when((t == 0) & (h == 0))
    def _build_logw():
        adjf = adj_ref[...].astype(jnp.float32)
        row_i = jax.lax.broadcasted_iota(jnp.int32, (_N, _N), 0)
        col_j = jax.lax.broadcasted_iota(jnp.int32, (_N, _N), 1)
        count = adjf + (row_i == col_j).astype(jnp.float32)
        logw_ref[...] = jnp.where(count == 0.0, -1e30, count - 1.0)

    x_t = x_ref[0]
    w_h = w_ref[...]
    hfeat = jax.lax.dot_general(
        x_t, w_h, (((1,), (1,)), ((), ())),
        preferred_element_type=jnp.float32)
    asrc_col = jax.lax.dot_general(
        hfeat, asrc_ref[0], (((1,), (1,)), ((), ())),
        preferred_element_type=jnp.float32)
    adst_row = jax.lax.dot_general(
        adst_ref[0], hfeat, (((1,), (1,)), ((), ())),
        preferred_element_type=jnp.float32)

    s = asrc_col + adst_row
    s = jnp.maximum(s, 0.2 * s)
    e = s + logw_ref[...]
    amax = jnp.max(e, axis=0, keepdims=True)
    p = jnp.exp2(e - amax)

    contrib = jax.lax.dot_general(
        p, hfeat, (((0,), (0,)), ((), ())),
        preferred_element_type=jnp.float32)
    ones = jnp.ones((_N, 1), dtype=jnp.float32)
    denom = jax.lax.dot_general(
        p, ones, (((0,), (0,)), ((), ())),
        preferred_element_type=jnp.float32)
    contrib = contrib * ((1.0 / _H) / (denom + 1e-16))

    @pl.when(h == 0)
    def _init():
        out_ref[0] = bias_ref[...] + contrib

    @pl.when(h != 0)
    def _acc():
        out_ref[0] = out_ref[0] + contrib


def kernel(x, node_matrix, W, att_src, att_dst, bias):
    x_t_major = jnp.transpose(x, (1, 0, 2))
    bias2d = bias.reshape(1, _C)
    log2e = jnp.float32(1.4426950408889634)
    out = pl.pallas_call(
        _gat_kernel,
        grid=(_T, _H),
        in_specs=[
            pl.BlockSpec((1, _N, _IN_DIM), lambda t, h: (t, 0, 0)),
            pl.BlockSpec((_N, _N), lambda t, h: (0, 0)),
            pl.BlockSpec((_C, _IN_DIM), lambda t, h: (h, 0)),
            pl.BlockSpec((1, 1, _C), lambda t, h: (h, 0, 0)),
            pl.BlockSpec((1, 1, _C), lambda t, h: (h, 0, 0)),
            pl.BlockSpec((1, _C), lambda t, h: (0, 0)),
        ],
        out_specs=pl.BlockSpec((1, _N, _C), lambda t, h: (t, 0, 0)),
        out_shape=jax.ShapeDtypeStruct((_T, _N, _C), jnp.float32),
        scratch_shapes=[pltpu.VMEM((_N, _N), jnp.float32)],
    )(x_t_major, node_matrix, W,
      (att_src * log2e).reshape(_H, 1, _C),
      (att_dst * log2e).reshape(_H, 1, _C), bias2d)
    return jnp.transpose(out, (1, 0, 2))


# speedup vs baseline: 1.1917x; 1.1588x over previous
import jax
import jax.numpy as jnp
from jax.experimental import pallas as pl
from jax.experimental.pallas import tpu as pltpu

_N = 1024
_T = 4
_IN_DIM = 128
_C = 128
_H = 4


def _gat_kernel(x_ref, adj_ref, w_ref, asrc_ref, adst_ref, bias_ref,
                out_ref, logw_ref):
    t = pl.program_id(0)

    @pl.when(t == 0)
    def _build_logw():
        adjf = adj_ref[...].astype(jnp.float32)
        row_i = jax.lax.broadcasted_iota(jnp.int32, (_N, _N), 0)
        col_j = jax.lax.broadcasted_iota(jnp.int32, (_N, _N), 1)
        count = adjf + (row_i == col_j).astype(jnp.float32)
        logw_ref[...] = jnp.where(count == 0.0, -1e30, count - 1.0)

    x_t = x_ref[...]
    hall = jax.lax.dot_general(
        x_t, w_ref[...], (((1,), (1,)), ((), ())),
        preferred_element_type=jnp.float32)
    logw = logw_ref[...]
    ones = jnp.ones((_N, 1), dtype=jnp.float32)

    acc = None
    for h in range(_H):
        hf = hall[:, h * _C:(h + 1) * _C]
        asrc_col = jax.lax.dot_general(
            hf, asrc_ref[h], (((1,), (1,)), ((), ())),
            preferred_element_type=jnp.float32)
        adst_row = jax.lax.dot_general(
            adst_ref[h], hf, (((1,), (1,)), ((), ())),
            preferred_element_type=jnp.float32)
        s = asrc_col + adst_row
        s = jnp.maximum(s, 0.2 * s)
        e = s + logw
        amax = jnp.max(e, axis=0, keepdims=True)
        p = jnp.exp2(e - amax)
        contrib = jax.lax.dot_general(
            p, hf, (((0,), (0,)), ((), ())),
            preferred_element_type=jnp.float32)
        denom = jax.lax.dot_general(
            p, ones, (((0,), (0,)), ((), ())),
            preferred_element_type=jnp.float32)
        term = contrib * ((1.0 / _H) / (denom + 1e-16))
        acc = term if acc is None else acc + term

    out_ref[...] = acc + bias_ref[...]


def kernel(x, node_matrix, W, att_src, att_dst, bias):
    x2d = x.reshape(_N, _T * _IN_DIM)
    bias2d = bias.reshape(1, _C)
    log2e = jnp.float32(1.4426950408889634)
    out = pl.pallas_call(
        _gat_kernel,
        grid=(_T,),
        in_specs=[
            pl.BlockSpec((_N, _IN_DIM), lambda t: (0, t)),
            pl.BlockSpec((_N, _N), lambda t: (0, 0)),
            pl.BlockSpec((_H * _C, _IN_DIM), lambda t: (0, 0)),
            pl.BlockSpec((_H, 1, _C), lambda t: (0, 0, 0)),
            pl.BlockSpec((_H, 1, _C), lambda t: (0, 0, 0)),
            pl.BlockSpec((1, _C), lambda t: (0, 0)),
        ],
        out_specs=pl.BlockSpec((_N, _C), lambda t: (0, t)),
        out_shape=jax.ShapeDtypeStruct((_N, _T * _C), jnp.float32),
        scratch_shapes=[pltpu.VMEM((_N, _N), jnp.float32)],
    )(x2d, node_matrix, W,
      (att_src * log2e).reshape(_H, 1, _C),
      (att_dst * log2e).reshape(_H, 1, _C), bias2d)
    return out.reshape(_N, _T, _C)
